# baseline (speedup 1.0000x reference)
import functools

import jax
import jax.numpy as jnp
from jax import lax
from jax.experimental import pallas as pl
from jax.experimental.pallas import tpu as pltpu
from jax.experimental.pallas import tpu_sc as plsc

_N, _D, _M = 320000, 128, 10000
_L = 16
_NT = 16
_EV = _N // _NT
_B = 25
_MP = 10240
_CS = _MP // _NT
_MAIN = 624
_CH = 48
_RB = _EV + 2 * _L


def _repair(tab, idx, val):
    def rbody(r, _):
        c = plsc.load_gather(tab, [idx])
        u = val > c
        plsc.store_scatter(tab, [idx], val, mask=u)
        return 0

    lax.fori_loop(0, _L, rbody, 0)


def _scatter_max_batch(tab, idxs, vals, poss, retry, roff):
    curs = [plsc.load_gather(tab, [i]) for i in idxs]
    upds = [v > c for v, c in zip(vals, curs)]
    for i, v, u in zip(idxs, vals, upds):
        plsc.store_scatter(tab, [i], v, mask=u)
    backs = [plsc.load_gather(tab, [i], mask=u) for i, u in zip(idxs, upds)]
    for p, v, u, b in zip(poss, vals, upds, backs):
        badv = u & (b != v)
        plsc.store_compressed(retry.at[pl.ds(roff, _L)], p, mask=badv)
        roff = roff + plsc.all_reduce_population_count(badv)[0]
    return roff


def _replay(tab, retry, roff, ev_idx, value_fn):
    zeros = jnp.zeros((_L,), jnp.int32)
    plsc.store_compressed(retry.at[pl.ds(roff, _L)], zeros,
                          mask=jnp.full((_L,), True))

    def rv(k, _):
        pos = retry[pl.ds(k * _L, _L)]
        idx = plsc.load_gather(ev_idx, [pos])
        _repair(tab, idx, value_fn(pos))
        return 0

    lax.fori_loop(0, (roff + _L - 1) // _L, rv, 0)


def _merge_tables(tab, stage, merged, col, blkbuf, blkbuf2, accb, msem, msem2):
    pltpu.sync_copy(tab, stage.at[col // _CS])
    plsc.subcore_barrier()
    pltpu.sync_copy(stage.at[0, pl.ds(col, _CS)], accb)
    bufs = [blkbuf, blkbuf2]
    sems = [msem, msem2]
    cps = [None] * _NT
    cps[1] = pltpu.async_copy(stage.at[1, pl.ds(col, _CS)], bufs[1], sems[1])
    for r in range(1, _NT):
        if r + 1 < _NT:
            cps[r + 1] = pltpu.async_copy(
                stage.at[r + 1, pl.ds(col, _CS)], bufs[(r + 1) % 2],
                sems[(r + 1) % 2])
        cps[r].wait()
        buf = bufs[r % 2]

        def ubody(u, _):
            sl = pl.ds(u * _L, _L)
            accb[sl] = jnp.maximum(accb[sl], buf[sl])
            return 0

        lax.fori_loop(0, _CS // _L, ubody, 0)
    if merged is not None:
        pltpu.sync_copy(accb, merged.at[pl.ds(col, _CS)])
        plsc.subcore_barrier()
        pltpu.sync_copy(merged, tab)


def _build_kernel():
    mesh = plsc.VectorSubcoreMesh(core_axis_name="c", subcore_axis_name="s")

    @functools.partial(
        pl.kernel,
        out_type=[
            jax.ShapeDtypeStruct((_M,), jnp.int32),
            jax.ShapeDtypeStruct((_M, _D), jnp.float32),
        ],
        mesh=mesh,
        compiler_params=pltpu.CompilerParams(needs_layout_passes=False),
        scratch_types=[
            pltpu.VMEM((_EV,), jnp.int32),
            pltpu.VMEM((_EV,), jnp.int32),
            pltpu.VMEM((_MP,), jnp.int32),
            pltpu.VMEM((_MP,), jnp.int32),
            pltpu.VMEM((_MP,), jnp.int32),
            pltpu.VMEM((_MP,), jnp.int32),
            pltpu.VMEM((_CS,), jnp.int32),
            pltpu.VMEM((_CS,), jnp.int32),
            pltpu.VMEM((_CS,), jnp.int32),
            pltpu.VMEM((_RB,), jnp.int32),
            pltpu.VMEM((_MAIN + _L,), jnp.int32),
            pltpu.VMEM((_MAIN + _L,), jnp.int32),
            pltpu.VMEM((_NT * _L,), jnp.int32),
            pltpu.VMEM((2, _CH, _D), jnp.float32),
            pltpu.VMEM_SHARED((_NT, _MP), jnp.int32),
            pltpu.VMEM_SHARED((_MP,), jnp.int32),
            pltpu.SemaphoreType.DMA,
            pltpu.SemaphoreType.DMA,
        ],
    )
    def lastagg(msg_hbm, idx_hbm, t_hbm, uniq_hbm, out_hbm,
                ev_idx, ev_t, maxt, argt, uniqv, srcv, blkbuf, blkbuf2,
                accb, retry, idxb, ub, cntb, rowb, stage, merged,
                sem, sem2):
        cid = lax.axis_index("c")
        sid = lax.axis_index("s")

        @pl.when(cid == 0)
        def _core0():
            lane = lax.iota(jnp.int32, _L)
            neg1 = jnp.full((_L,), -1, jnp.int32)
            col = sid * _CS

            def ibody(u, _):
                sl = pl.ds(u * _L, _L)
                maxt[sl] = neg1
                argt[sl] = neg1
                return 0

            base = pl.multiple_of(sid * _EV, 8)
            cpi = pltpu.async_copy(idx_hbm.at[pl.ds(base, _EV)], ev_idx, sem)
            cpt = pltpu.async_copy(t_hbm.at[pl.ds(base, _EV)], ev_t, sem2)
            lax.fori_loop(0, _MP // _L, ibody, 0)
            cpi.wait()
            cpt.wait()

            def p1(v, roff):
                sls = [pl.ds((v * _B + j) * _L, _L) for j in range(_B)]
                poss = [jnp.full((_L,), (v * _B + j) * _L, jnp.int32) + lane
                        for j in range(_B)]
                return _scatter_max_batch(maxt,
                                          [ev_idx[s] for s in sls],
                                          [ev_t[s] for s in sls],
                                          poss, retry, roff)

            r1 = lax.fori_loop(0, _EV // (_L * _B), p1, jnp.int32(0))
            _replay(maxt, retry, r1, ev_idx,
                    lambda pos: plsc.load_gather(ev_t, [pos]))
            _merge_tables(maxt, stage, merged, col, blkbuf, blkbuf2, accb, sem, sem2)

            def p2cand(pos):
                ix = plsc.load_gather(ev_idx, [pos])
                tv = plsc.load_gather(ev_t, [pos])
                gm = plsc.load_gather(maxt, [ix])
                gid = jnp.full((_L,), sid * _EV, jnp.int32) + pos
                return jnp.where(tv == gm, gid, neg1)

            def p2(v, roff):
                sls = [pl.ds((v * _B + j) * _L, _L) for j in range(_B)]
                poss = [jnp.full((_L,), (v * _B + j) * _L, jnp.int32) + lane
                        for j in range(_B)]
                idxs = [ev_idx[s] for s in sls]
                cands = []
                for j, (s, ix) in enumerate(zip(sls, idxs)):
                    tv = ev_t[s]
                    gm = plsc.load_gather(maxt, [ix])
                    gid = jnp.full((_L,), sid * _EV + (v * _B + j) * _L,
                                   jnp.int32) + lane
                    cands.append(jnp.where(tv == gm, gid, neg1))
                return _scatter_max_batch(argt, idxs, cands, poss, retry, roff)

            r2 = lax.fori_loop(0, _EV // (_L * _B), p2, jnp.int32(0))
            _replay(argt, retry, r2, ev_idx, p2cand)
            _merge_tables(argt, stage, None, col, blkbuf, blkbuf2, accb, sem, sem2)

            def cbody(u, cnt):
                sl = pl.ds(col + u * _L, _L)
                pres = maxt[sl] >= 0
                ids = jnp.full((_L,), col + u * _L, jnp.int32) + lane
                plsc.store_compressed(uniqv.at[pl.ds(col + cnt, _L)], ids,
                                      mask=pres)
                plsc.store_compressed(srcv.at[pl.ds(col + cnt, _L)],
                                      accb[pl.ds(u * _L, _L)], mask=pres)
                return cnt + plsc.all_reduce_population_count(pres)[0]

            cnt = lax.fori_loop(0, _CS // _L, cbody, jnp.int32(0))
            pltpu.sync_copy(uniqv.at[pl.ds(col, _CS)],
                            stage.at[0, pl.ds(col, _CS)])
            pltpu.sync_copy(srcv.at[pl.ds(col, _CS)],
                            stage.at[1, pl.ds(col, _CS)])
            accb[pl.ds(0, _L)] = jnp.full((_L,), cnt, jnp.int32)
            pltpu.sync_copy(accb.at[pl.ds(0, _L)],
                            stage.at[2, pl.ds(sid * _L, _L)])
            plsc.subcore_barrier()
            pltpu.sync_copy(stage.at[0], uniqv)
            pltpu.sync_copy(stage.at[1], srcv)
            pltpu.sync_copy(stage.at[2, pl.ds(0, _NT * _L)], cntb)
            counts = plsc.load_gather(cntb, [lane * _L])
            pinc = plsc.cumsum(counts)
            pex = pinc - counts
            ktot = jnp.max(pinc)

            def obody(u, _):
                p = jnp.full((_L,), sid * _MAIN + u * _L, jnp.int32) + lane
                peff = jnp.where(p < ktot, p, 0)
                sslot = jnp.zeros((_L,), jnp.int32)
                for r in range(1, _NT):
                    thr = pex.at[jnp.full((_L,), r, jnp.int32)].get(
                        mode="promise_in_bounds")
                    sslot = sslot + (thr <= peff).astype(jnp.int32)
                psel = pex.at[sslot].get(mode="promise_in_bounds")
                q = sslot * _CS + (peff - psel)
                sl = pl.ds(u * _L, _L)
                ub[sl] = plsc.load_gather(uniqv, [q])
                idxb[sl] = plsc.load_gather(srcv, [q])
                return 0

            lax.fori_loop(0, _MAIN // _L, obody, 0)
            rbase = pl.multiple_of(sid * _MAIN, 8)
            pltpu.sync_copy(ub.at[pl.ds(0, _MAIN)],
                            uniq_hbm.at[pl.ds(rbase, _MAIN)])

            @pl.when(sid == _NT - 1)
            def _tailmap():
                u = _MAIN // _L
                p = jnp.full((_L,), _NT * _MAIN, jnp.int32) + lane
                peff = jnp.where(p < ktot, p, 0)
                sslot = jnp.zeros((_L,), jnp.int32)
                for r in range(1, _NT):
                    thr = pex.at[jnp.full((_L,), r, jnp.int32)].get(
                        mode="promise_in_bounds")
                    sslot = sslot + (thr <= peff).astype(jnp.int32)
                psel = pex.at[sslot].get(mode="promise_in_bounds")
                q = sslot * _CS + (peff - psel)
                sl = pl.ds(_MAIN, _L)
                ub[sl] = plsc.load_gather(uniqv, [q])
                idxb[sl] = plsc.load_gather(srcv, [q])
                toff = pl.multiple_of(_NT * _MAIN, 8)
                pltpu.sync_copy(ub.at[sl], uniq_hbm.at[pl.ds(toff, _L)])

            nch = _MAIN // _CH
            sems = [sem, sem2]
            cps = [None] * nch
            cps[0] = pltpu.async_copy(msg_hbm.at[idxb.at[pl.ds(0, _CH)]],
                                      rowb.at[0], sems[0])
            for k in range(nch):
                if k + 1 < nch:
                    cps[k + 1] = pltpu.async_copy(
                        msg_hbm.at[idxb.at[pl.ds((k + 1) * _CH, _CH)]],
                        rowb.at[(k + 1) % 2], sems[(k + 1) % 2])
                cps[k].wait()
                pltpu.sync_copy(rowb.at[k % 2],
                                out_hbm.at[pl.ds(rbase + k * _CH, _CH)])

            @pl.when(sid == _NT - 1)
            def _tail():
                toff = pl.multiple_of(_NT * _MAIN, 8)
                pltpu.async_copy(msg_hbm.at[idxb.at[pl.ds(_MAIN, _L)]],
                                 rowb.at[0, pl.ds(0, _L)], sem).wait()
                pltpu.sync_copy(rowb.at[0, pl.ds(0, _L)],
                                out_hbm.at[pl.ds(toff, _L)])

    return lastagg


_lastagg = _build_kernel()


@jax.jit
def kernel(msg, index, t):
    uniq, rows = _lastagg(msg, index, t)
    return uniq, rows


# speedup vs baseline: 1.1135x; 1.1135x over previous
import functools

import jax
import jax.numpy as jnp
from jax import lax
from jax.experimental import pallas as pl
from jax.experimental.pallas import tpu as pltpu
from jax.experimental.pallas import tpu_sc as plsc

_N, _D, _M = 320000, 128, 10000
_L = 16
_NT = 16
_EV = _N // _NT
_B = 10
_MP = 10240
_CS = _MP // _NT
_MAIN = 624
_CH = 48
_RB = _EV + 2 * _L


def _repair(tab, idx, val):
    def rbody(r, _):
        c = plsc.load_gather(tab, [idx])
        u = val > c
        plsc.store_scatter(tab, [idx], val, mask=u)
        return 0

    lax.fori_loop(0, _L, rbody, 0)


def _scatter_max_batch(tab, idxs, vals, poss, retry, roff):
    curs = [plsc.load_gather(tab, [i]) for i in idxs]
    upds = [v > c for v, c in zip(vals, curs)]
    for i, v, u in zip(idxs, vals, upds):
        plsc.store_scatter(tab, [i], v, mask=u)
    backs = [plsc.load_gather(tab, [i], mask=u) for i, u in zip(idxs, upds)]
    for p, v, u, b in zip(poss, vals, upds, backs):
        badv = u & (b != v)
        plsc.store_compressed(retry.at[pl.ds(roff, _L)], p, mask=badv)
        roff = roff + plsc.all_reduce_population_count(badv)[0]
    return roff


def _replay(tab, retry, roff, ev_idx, value_fn):
    zeros = jnp.zeros((_L,), jnp.int32)
    plsc.store_compressed(retry.at[pl.ds(roff, _L)], zeros,
                          mask=jnp.full((_L,), True))

    def rv(k, _):
        pos = retry[pl.ds(k * _L, _L)]
        idx = plsc.load_gather(ev_idx, [pos])
        _repair(tab, idx, value_fn(pos))
        return 0

    lax.fori_loop(0, (roff + _L - 1) // _L, rv, 0)


def _merge_tables(tab, stage, merged, col, blkbuf, blkbuf2, accb, msem, msem2):
    pltpu.sync_copy(tab, stage.at[col // _CS])
    plsc.subcore_barrier()
    pltpu.sync_copy(stage.at[0, pl.ds(col, _CS)], accb)
    bufs = [blkbuf, blkbuf2]
    sems = [msem, msem2]
    cps = [None] * _NT
    cps[1] = pltpu.async_copy(stage.at[1, pl.ds(col, _CS)], bufs[1], sems[1])
    for r in range(1, _NT):
        if r + 1 < _NT:
            cps[r + 1] = pltpu.async_copy(
                stage.at[r + 1, pl.ds(col, _CS)], bufs[(r + 1) % 2],
                sems[(r + 1) % 2])
        cps[r].wait()
        buf = bufs[r % 2]

        def ubody(u, _):
            sl = pl.ds(u * _L, _L)
            accb[sl] = jnp.maximum(accb[sl], buf[sl])
            return 0

        lax.fori_loop(0, _CS // _L, ubody, 0)
    if merged is not None:
        pltpu.sync_copy(accb, merged.at[pl.ds(col, _CS)])
        plsc.subcore_barrier()
        pltpu.sync_copy(merged, tab)


def _build_kernel():
    mesh = plsc.VectorSubcoreMesh(core_axis_name="c", subcore_axis_name="s")

    @functools.partial(
        pl.kernel,
        out_type=[
            jax.ShapeDtypeStruct((_M,), jnp.int32),
            jax.ShapeDtypeStruct((_M, _D), jnp.float32),
        ],
        mesh=mesh,
        compiler_params=pltpu.CompilerParams(needs_layout_passes=False),
        scratch_types=[
            pltpu.VMEM((_EV,), jnp.int32),
            pltpu.VMEM((_EV,), jnp.int32),
            pltpu.VMEM((_MP,), jnp.int32),
            pltpu.VMEM((_MP,), jnp.int32),
            pltpu.VMEM((_MP,), jnp.int32),
            pltpu.VMEM((_MP,), jnp.int32),
            pltpu.VMEM((_CS,), jnp.int32),
            pltpu.VMEM((_CS,), jnp.int32),
            pltpu.VMEM((_CS,), jnp.int32),
            pltpu.VMEM((_RB,), jnp.int32),
            pltpu.VMEM((_MAIN + _L,), jnp.int32),
            pltpu.VMEM((_MAIN + _L,), jnp.int32),
            pltpu.VMEM((_NT * _L,), jnp.int32),
            pltpu.VMEM((2, _CH, _D), jnp.float32),
            pltpu.VMEM_SHARED((_NT, _MP), jnp.int32),
            pltpu.VMEM_SHARED((_MP,), jnp.int32),
            pltpu.SemaphoreType.DMA,
            pltpu.SemaphoreType.DMA,
        ],
    )
    def lastagg(msg_hbm, idx_hbm, t_hbm, uniq_hbm, out_hbm,
                ev_idx, ev_t, maxt, argt, uniqv, srcv, blkbuf, blkbuf2,
                accb, retry, idxb, ub, cntb, rowb, stage, merged,
                sem, sem2):
        cid = lax.axis_index("c")
        sid = lax.axis_index("s")

        @pl.when(cid == 0)
        def _core0():
            lane = lax.iota(jnp.int32, _L)
            neg1 = jnp.full((_L,), -1, jnp.int32)
            col = sid * _CS

            def ibody(u, _):
                sl = pl.ds(u * _L, _L)
                maxt[sl] = neg1
                argt[sl] = neg1
                return 0

            base = pl.multiple_of(sid * _EV, 8)
            cpi = pltpu.async_copy(idx_hbm.at[pl.ds(base, _EV)], ev_idx, sem)
            cpt = pltpu.async_copy(t_hbm.at[pl.ds(base, _EV)], ev_t, sem2)
            lax.fori_loop(0, _MP // _L, ibody, 0)
            cpi.wait()
            cpt.wait()

            def p1(v, roff):
                sls = [pl.ds((v * _B + j) * _L, _L) for j in range(_B)]
                poss = [jnp.full((_L,), (v * _B + j) * _L, jnp.int32) + lane
                        for j in range(_B)]
                return _scatter_max_batch(maxt,
                                          [ev_idx[s] for s in sls],
                                          [ev_t[s] for s in sls],
                                          poss, retry, roff)

            r1 = lax.fori_loop(0, _EV // (_L * _B), p1, jnp.int32(0))
            _replay(maxt, retry, r1, ev_idx,
                    lambda pos: plsc.load_gather(ev_t, [pos]))
            _merge_tables(maxt, stage, merged, col, blkbuf, blkbuf2, accb, sem, sem2)

            def p2cand(pos):
                ix = plsc.load_gather(ev_idx, [pos])
                tv = plsc.load_gather(ev_t, [pos])
                gm = plsc.load_gather(maxt, [ix])
                gid = jnp.full((_L,), sid * _EV, jnp.int32) + pos
                return jnp.where(tv == gm, gid, neg1)

            def p2(v, roff):
                sls = [pl.ds((v * _B + j) * _L, _L) for j in range(_B)]
                poss = [jnp.full((_L,), (v * _B + j) * _L, jnp.int32) + lane
                        for j in range(_B)]
                idxs = [ev_idx[s] for s in sls]
                cands = []
                for j, (s, ix) in enumerate(zip(sls, idxs)):
                    tv = ev_t[s]
                    gm = plsc.load_gather(maxt, [ix])
                    gid = jnp.full((_L,), sid * _EV + (v * _B + j) * _L,
                                   jnp.int32) + lane
                    cands.append(jnp.where(tv == gm, gid, neg1))
                return _scatter_max_batch(argt, idxs, cands, poss, retry, roff)

            r2 = lax.fori_loop(0, _EV // (_L * _B), p2, jnp.int32(0))
            _replay(argt, retry, r2, ev_idx, p2cand)
            _merge_tables(argt, stage, None, col, blkbuf, blkbuf2, accb, sem, sem2)

            def cbody(u, cnt):
                sl = pl.ds(col + u * _L, _L)
                pres = maxt[sl] >= 0
                ids = jnp.full((_L,), col + u * _L, jnp.int32) + lane
                plsc.store_compressed(uniqv.at[pl.ds(col + cnt, _L)], ids,
                                      mask=pres)
                plsc.store_compressed(srcv.at[pl.ds(col + cnt, _L)],
                                      accb[pl.ds(u * _L, _L)], mask=pres)
                return cnt + plsc.all_reduce_population_count(pres)[0]

            cnt = lax.fori_loop(0, _CS // _L, cbody, jnp.int32(0))
            pltpu.sync_copy(uniqv.at[pl.ds(col, _CS)],
                            stage.at[0, pl.ds(col, _CS)])
            pltpu.sync_copy(srcv.at[pl.ds(col, _CS)],
                            stage.at[1, pl.ds(col, _CS)])
            accb[pl.ds(0, _L)] = jnp.full((_L,), cnt, jnp.int32)
            pltpu.sync_copy(accb.at[pl.ds(0, _L)],
                            stage.at[2, pl.ds(sid * _L, _L)])
            plsc.subcore_barrier()
            pltpu.sync_copy(stage.at[0], uniqv)
            pltpu.sync_copy(stage.at[1], srcv)
            pltpu.sync_copy(stage.at[2, pl.ds(0, _NT * _L)], cntb)
            counts = plsc.load_gather(cntb, [lane * _L])
            pinc = plsc.cumsum(counts)
            pex = pinc - counts
            ktot = jnp.max(pinc)

            def obody(u, _):
                p = jnp.full((_L,), sid * _MAIN + u * _L, jnp.int32) + lane
                peff = jnp.where(p < ktot, p, 0)
                sslot = jnp.zeros((_L,), jnp.int32)
                for r in range(1, _NT):
                    thr = pex.at[jnp.full((_L,), r, jnp.int32)].get(
                        mode="promise_in_bounds")
                    sslot = sslot + (thr <= peff).astype(jnp.int32)
                psel = pex.at[sslot].get(mode="promise_in_bounds")
                q = sslot * _CS + (peff - psel)
                sl = pl.ds(u * _L, _L)
                ub[sl] = plsc.load_gather(uniqv, [q])
                idxb[sl] = plsc.load_gather(srcv, [q])
                return 0

            lax.fori_loop(0, _MAIN // _L, obody, 0)
            rbase = pl.multiple_of(sid * _MAIN, 8)
            pltpu.sync_copy(ub.at[pl.ds(0, _MAIN)],
                            uniq_hbm.at[pl.ds(rbase, _MAIN)])

            @pl.when(sid == _NT - 1)
            def _tailmap():
                u = _MAIN // _L
                p = jnp.full((_L,), _NT * _MAIN, jnp.int32) + lane
                peff = jnp.where(p < ktot, p, 0)
                sslot = jnp.zeros((_L,), jnp.int32)
                for r in range(1, _NT):
                    thr = pex.at[jnp.full((_L,), r, jnp.int32)].get(
                        mode="promise_in_bounds")
                    sslot = sslot + (thr <= peff).astype(jnp.int32)
                psel = pex.at[sslot].get(mode="promise_in_bounds")
                q = sslot * _CS + (peff - psel)
                sl = pl.ds(_MAIN, _L)
                ub[sl] = plsc.load_gather(uniqv, [q])
                idxb[sl] = plsc.load_gather(srcv, [q])
                toff = pl.multiple_of(_NT * _MAIN, 8)
                pltpu.sync_copy(ub.at[sl], uniq_hbm.at[pl.ds(toff, _L)])

            nch = _MAIN // _CH
            sems = [sem, sem2]
            cps = [None] * nch
            cps[0] = pltpu.async_copy(msg_hbm.at[idxb.at[pl.ds(0, _CH)]],
                                      rowb.at[0], sems[0])
            for k in range(nch):
                if k + 1 < nch:
                    cps[k + 1] = pltpu.async_copy(
                        msg_hbm.at[idxb.at[pl.ds((k + 1) * _CH, _CH)]],
                        rowb.at[(k + 1) % 2], sems[(k + 1) % 2])
                cps[k].wait()
                pltpu.sync_copy(rowb.at[k % 2],
                                out_hbm.at[pl.ds(rbase + k * _CH, _CH)])

            @pl.when(sid == _NT - 1)
            def _tail():
                toff = pl.multiple_of(_NT * _MAIN, 8)
                pltpu.async_copy(msg_hbm.at[idxb.at[pl.ds(_MAIN, _L)]],
                                 rowb.at[0, pl.ds(0, _L)], sem).wait()
                pltpu.sync_copy(rowb.at[0, pl.ds(0, _L)],
                                out_hbm.at[pl.ds(toff, _L)])

    return lastagg


_lastagg = _build_kernel()


@jax.jit
def kernel(msg, index, t):
    uniq, rows = _lastagg(msg, index, t)
    return uniq, rows


# speedup vs baseline: 1.1160x; 1.0023x over previous
import functools

import jax
import jax.numpy as jnp
from jax import lax
from jax.experimental import pallas as pl
from jax.experimental.pallas import tpu as pltpu
from jax.experimental.pallas import tpu_sc as plsc

_N, _D, _M = 320000, 128, 10000
_L = 16
_NT = 16
_EV = _N // _NT
_B = 10
_MP = 10240
_CS = _MP // _NT
_MAIN = 624
_CH = 48
_RB = _EV + 2 * _L


def _repair(tab, idx, val):
    def rbody(r, _):
        c = plsc.load_gather(tab, [idx])
        u = val > c
        plsc.store_scatter(tab, [idx], val, mask=u)
        return 0

    lax.fori_loop(0, _L, rbody, 0)


def _scatter_max_batch(tab, idxs, vals, poss, retry, roff):
    curs = [plsc.load_gather(tab, [i]) for i in idxs]
    upds = [v > c for v, c in zip(vals, curs)]
    for i, v, u in zip(idxs, vals, upds):
        plsc.store_scatter(tab, [i], v, mask=u)
    backs = [plsc.load_gather(tab, [i], mask=u) for i, u in zip(idxs, upds)]
    for p, v, u, b in zip(poss, vals, upds, backs):
        badv = u & (b != v)
        plsc.store_compressed(retry.at[pl.ds(roff, _L)], p, mask=badv)
        roff = roff + plsc.all_reduce_population_count(badv)[0]
    return roff


def _replay(tab, retry, roff, ev_idx, value_fn):
    zeros = jnp.zeros((_L,), jnp.int32)
    plsc.store_compressed(retry.at[pl.ds(roff, _L)], zeros,
                          mask=jnp.full((_L,), True))

    def rv(k, _):
        pos = retry[pl.ds(k * _L, _L)]
        idx = plsc.load_gather(ev_idx, [pos])
        _repair(tab, idx, value_fn(pos))
        return 0

    lax.fori_loop(0, (roff + _L - 1) // _L, rv, 0)


def _merge_tables(tab, stage, merged, col, blkbuf, blkbuf2, accb, msem, msem2):
    pltpu.sync_copy(tab, stage.at[col // _CS])
    plsc.subcore_barrier()
    pltpu.sync_copy(stage.at[0, pl.ds(col, _CS)], accb)
    bufs = [blkbuf, blkbuf2]
    sems = [msem, msem2]
    cps = [None] * _NT
    cps[1] = pltpu.async_copy(stage.at[1, pl.ds(col, _CS)], bufs[1], sems[1])
    for r in range(1, _NT):
        if r + 1 < _NT:
            cps[r + 1] = pltpu.async_copy(
                stage.at[r + 1, pl.ds(col, _CS)], bufs[(r + 1) % 2],
                sems[(r + 1) % 2])
        cps[r].wait()
        buf = bufs[r % 2]

        def ubody(u, _):
            sl = pl.ds(u * _L, _L)
            accb[sl] = jnp.maximum(accb[sl], buf[sl])
            return 0

        lax.fori_loop(0, _CS // _L, ubody, 0)
    if merged is not None:
        pltpu.sync_copy(accb, merged.at[pl.ds(col, _CS)])
        plsc.subcore_barrier()
        pltpu.sync_copy(merged, tab)


def _build_kernel():
    mesh = plsc.VectorSubcoreMesh(core_axis_name="c", subcore_axis_name="s")

    @functools.partial(
        pl.kernel,
        out_type=[
            jax.ShapeDtypeStruct((_M,), jnp.int32),
            jax.ShapeDtypeStruct((_M, _D), jnp.float32),
        ],
        mesh=mesh,
        compiler_params=pltpu.CompilerParams(needs_layout_passes=False),
        scratch_types=[
            pltpu.VMEM((_EV,), jnp.int32),
            pltpu.VMEM((_EV,), jnp.int32),
            pltpu.VMEM((_MP,), jnp.int32),
            pltpu.VMEM((_MP,), jnp.int32),
            pltpu.VMEM((_MP,), jnp.int32),
            pltpu.VMEM((_MP,), jnp.int32),
            pltpu.VMEM((_CS,), jnp.int32),
            pltpu.VMEM((_CS,), jnp.int32),
            pltpu.VMEM((_CS,), jnp.int32),
            pltpu.VMEM((_RB,), jnp.int32),
            pltpu.VMEM((_MAIN + _L,), jnp.int32),
            pltpu.VMEM((_MAIN + _L,), jnp.int32),
            pltpu.VMEM((_NT * _L,), jnp.int32),
            pltpu.VMEM((2, _CH, _D), jnp.float32),
            pltpu.VMEM_SHARED((_NT, _MP), jnp.int32),
            pltpu.VMEM_SHARED((_MP,), jnp.int32),
            pltpu.SemaphoreType.DMA,
            pltpu.SemaphoreType.DMA,
        ],
    )
    def lastagg(msg_hbm, idx_hbm, t_hbm, uniq_hbm, out_hbm,
                ev_idx, ev_t, maxt, argt, uniqv, srcv, blkbuf, blkbuf2,
                accb, retry, idxb, ub, cntb, rowb, stage, merged,
                sem, sem2):
        cid = lax.axis_index("c")
        sid = lax.axis_index("s")

        @pl.when(cid == 0)
        def _core0():
            lane = lax.iota(jnp.int32, _L)
            neg1 = jnp.full((_L,), -1, jnp.int32)
            col = sid * _CS

            def ibody(u, _):
                sl = pl.ds(u * _L, _L)
                maxt[sl] = neg1
                argt[sl] = neg1
                return 0

            base = pl.multiple_of(sid * _EV, 8)
            cpi = pltpu.async_copy(idx_hbm.at[pl.ds(base, _EV)], ev_idx, sem)
            cpt = pltpu.async_copy(t_hbm.at[pl.ds(base, _EV)], ev_t, sem2)
            lax.fori_loop(0, _MP // _L, ibody, 0)
            cpi.wait()
            cpt.wait()

            def p1(v, roff):
                sls = [pl.ds((v * _B + j) * _L, _L) for j in range(_B)]
                poss = [jnp.full((_L,), (v * _B + j) * _L, jnp.int32) + lane
                        for j in range(_B)]
                return _scatter_max_batch(maxt,
                                          [ev_idx[s] for s in sls],
                                          [ev_t[s] for s in sls],
                                          poss, retry, roff)

            r1 = lax.fori_loop(0, _EV // (_L * _B), p1, jnp.int32(0))
            _replay(maxt, retry, r1, ev_idx,
                    lambda pos: plsc.load_gather(ev_t, [pos]))
            _merge_tables(maxt, stage, merged, col, blkbuf, blkbuf2, accb, sem, sem2)

            def p2cand(pos):
                ix = plsc.load_gather(ev_idx, [pos])
                tv = plsc.load_gather(ev_t, [pos])
                gm = plsc.load_gather(maxt, [ix])
                gid = jnp.full((_L,), sid * _EV, jnp.int32) + pos
                return jnp.where(tv == gm, gid, neg1)

            def p2(v, roff):
                sls = [pl.ds((v * _B + j) * _L, _L) for j in range(_B)]
                poss = [jnp.full((_L,), (v * _B + j) * _L, jnp.int32) + lane
                        for j in range(_B)]
                idxs = [ev_idx[s] for s in sls]
                cands = []
                for j, (s, ix) in enumerate(zip(sls, idxs)):
                    tv = ev_t[s]
                    gm = plsc.load_gather(maxt, [ix])
                    gid = jnp.full((_L,), sid * _EV + (v * _B + j) * _L,
                                   jnp.int32) + lane
                    cands.append(jnp.where(tv == gm, gid, neg1))
                return _scatter_max_batch(argt, idxs, cands, poss, retry, roff)

            r2 = lax.fori_loop(0, _EV // (_L * _B), p2, jnp.int32(0))
            _replay(argt, retry, r2, ev_idx, p2cand)
            _merge_tables(argt, stage, None, col, blkbuf, blkbuf2, accb, sem, sem2)

            def cbody(u, cnt):
                sl = pl.ds(col + u * _L, _L)
                pres = maxt[sl] >= 0
                ids = jnp.full((_L,), col + u * _L, jnp.int32) + lane
                plsc.store_compressed(uniqv.at[pl.ds(col + cnt, _L)], ids,
                                      mask=pres)
                plsc.store_compressed(srcv.at[pl.ds(col + cnt, _L)],
                                      accb[pl.ds(u * _L, _L)], mask=pres)
                return cnt + plsc.all_reduce_population_count(pres)[0]

            cnt = lax.fori_loop(0, _CS // _L, cbody, jnp.int32(0))
            pltpu.sync_copy(uniqv.at[pl.ds(col, _CS)],
                            stage.at[0, pl.ds(col, _CS)])
            pltpu.sync_copy(srcv.at[pl.ds(col, _CS)],
                            stage.at[1, pl.ds(col, _CS)])
            accb[pl.ds(0, _L)] = jnp.full((_L,), cnt, jnp.int32)
            pltpu.sync_copy(accb.at[pl.ds(0, _L)],
                            merged.at[pl.ds(sid * _L, _L)])
            plsc.subcore_barrier()
            pltpu.sync_copy(stage.at[0], uniqv)
            pltpu.sync_copy(stage.at[1], srcv)
            pltpu.sync_copy(merged.at[pl.ds(0, _NT * _L)], cntb)
            counts = plsc.load_gather(cntb, [lane * _L])
            pinc = plsc.cumsum(counts)
            pex = pinc - counts
            ktot = jnp.max(pinc)

            def obody(u, _):
                p = jnp.full((_L,), sid * _MAIN + u * _L, jnp.int32) + lane
                peff = jnp.where(p < ktot, p, 0)
                sslot = jnp.zeros((_L,), jnp.int32)
                for r in range(1, _NT):
                    thr = pex.at[jnp.full((_L,), r, jnp.int32)].get(
                        mode="promise_in_bounds")
                    sslot = sslot + (thr <= peff).astype(jnp.int32)
                psel = pex.at[sslot].get(mode="promise_in_bounds")
                q = sslot * _CS + (peff - psel)
                sl = pl.ds(u * _L, _L)
                ub[sl] = plsc.load_gather(uniqv, [q])
                idxb[sl] = plsc.load_gather(srcv, [q])
                return 0

            lax.fori_loop(0, _MAIN // _L, obody, 0)
            rbase = pl.multiple_of(sid * _MAIN, 8)
            pltpu.sync_copy(ub.at[pl.ds(0, _MAIN)],
                            uniq_hbm.at[pl.ds(rbase, _MAIN)])

            @pl.when(sid == _NT - 1)
            def _tailmap():
                u = _MAIN // _L
                p = jnp.full((_L,), _NT * _MAIN, jnp.int32) + lane
                peff = jnp.where(p < ktot, p, 0)
                sslot = jnp.zeros((_L,), jnp.int32)
                for r in range(1, _NT):
                    thr = pex.at[jnp.full((_L,), r, jnp.int32)].get(
                        mode="promise_in_bounds")
                    sslot = sslot + (thr <= peff).astype(jnp.int32)
                psel = pex.at[sslot].get(mode="promise_in_bounds")
                q = sslot * _CS + (peff - psel)
                sl = pl.ds(_MAIN, _L)
                ub[sl] = plsc.load_gather(uniqv, [q])
                idxb[sl] = plsc.load_gather(srcv, [q])
                toff = pl.multiple_of(_NT * _MAIN, 8)
                pltpu.sync_copy(ub.at[sl], uniq_hbm.at[pl.ds(toff, _L)])

            nch = _MAIN // _CH
            sems = [sem, sem2]
            cps = [None] * nch
            cps[0] = pltpu.async_copy(msg_hbm.at[idxb.at[pl.ds(0, _CH)]],
                                      rowb.at[0], sems[0])
            for k in range(nch):
                if k + 1 < nch:
                    cps[k + 1] = pltpu.async_copy(
                        msg_hbm.at[idxb.at[pl.ds((k + 1) * _CH, _CH)]],
                        rowb.at[(k + 1) % 2], sems[(k + 1) % 2])
                cps[k].wait()
                pltpu.sync_copy(rowb.at[k % 2],
                                out_hbm.at[pl.ds(rbase + k * _CH, _CH)])

            @pl.when(sid == _NT - 1)
            def _tail():
                toff = pl.multiple_of(_NT * _MAIN, 8)
                pltpu.async_copy(msg_hbm.at[idxb.at[pl.ds(_MAIN, _L)]],
                                 rowb.at[0, pl.ds(0, _L)], sem).wait()
                pltpu.sync_copy(rowb.at[0, pl.ds(0, _L)],
                                out_hbm.at[pl.ds(toff, _L)])

    return lastagg


_lastagg = _build_kernel()


@jax.jit
def kernel(msg, index, t):
    uniq, rows = _lastagg(msg, index, t)
    return uniq, rows
